# initial kernel scaffold (unmeasured)
import jax
import jax.numpy as jnp
from jax import lax
from jax.experimental import pallas as pl
from jax.experimental.pallas import tpu as pltpu

N_DEV = 32
MP = 64
KP = 64


def kernel(x, w_mat):
    M, kp = x.shape
    K, N = w_mat.shape

    def body(x_ref, w_ref, out_ref, xbf_ref, xg_ref, send_sems, recv_sems):
        my = lax.axis_index("i")

        xbf_ref[:, :] = x_ref[:, :].astype(jnp.bfloat16)

        local = pltpu.make_async_copy(
            xbf_ref.at[pl.ds(my * MP, MP), :],
            xg_ref.at[my],
            recv_sems.at[my],
        )
        local.start()

        sends = []
        for d in range(1, N_DEV):
            t = (my + d) % N_DEV
            rdma = pltpu.make_async_remote_copy(
                src_ref=xbf_ref.at[pl.ds(t * MP, MP), :],
                dst_ref=xg_ref.at[my],
                send_sem=send_sems.at[d],
                recv_sem=recv_sems.at[my],
                device_id=(t,),
                device_id_type=pl.DeviceIdType.MESH,
            )
            rdma.start()
            sends.append(rdma)

        acc = jnp.zeros((MP, N), jnp.float32)
        for s in range(N_DEV):
            recv = pltpu.make_async_remote_copy(
                src_ref=xbf_ref.at[pl.ds(0, MP), :],
                dst_ref=xg_ref.at[s],
                send_sem=send_sems.at[0],
                recv_sem=recv_sems.at[s],
                device_id=(my,),
                device_id_type=pl.DeviceIdType.MESH,
            )
            recv.wait_recv()
            acc = acc + jnp.dot(
                xg_ref[s],
                w_ref[pl.ds(s * KP, KP), :].astype(jnp.bfloat16),
                preferred_element_type=jnp.float32,
            )

        for rdma in sends:
            rdma.wait_send()

        out_ref[:, :] = acc * jax.nn.sigmoid(acc)

    return pl.pallas_call(
        body,
        out_shape=jax.ShapeDtypeStruct((MP, N), jnp.float32),
        in_specs=[
            pl.BlockSpec(memory_space=pltpu.VMEM),
            pl.BlockSpec(memory_space=pltpu.VMEM),
        ],
        out_specs=pl.BlockSpec(memory_space=pltpu.VMEM),
        scratch_shapes=[
            pltpu.VMEM((M, kp), jnp.bfloat16),
            pltpu.VMEM((N_DEV, MP, KP), jnp.bfloat16),
            pltpu.SemaphoreType.DMA((N_DEV,)),
            pltpu.SemaphoreType.DMA((N_DEV,)),
        ],
        compiler_params=pltpu.CompilerParams(collective_id=0),
    )(x, w_mat)


# baseline (device time: 30476 ns/iter reference)
import jax
import jax.numpy as jnp
from jax import lax
from jax.experimental import pallas as pl
from jax.experimental.pallas import tpu as pltpu

N_DEV = 32
MP = 64
KP = 64


def kernel(x, w_mat):
    M, kp = x.shape
    K, N = w_mat.shape

    def body(x_ref, w_ref, out_ref, xbf_ref, xg_ref, send_sems, recv_sems):
        my = lax.axis_index("i")

        xbf_ref[:, :] = x_ref[:, :].astype(jnp.bfloat16)

        local = pltpu.make_async_copy(
            xbf_ref.at[pl.ds(my * MP, MP), :],
            xg_ref.at[my],
            recv_sems.at[my],
        )
        local.start()

        sends = []
        for d in range(1, N_DEV):
            t = (my + d) % N_DEV
            rdma = pltpu.make_async_remote_copy(
                src_ref=xbf_ref.at[pl.ds(t * MP, MP), :],
                dst_ref=xg_ref.at[my],
                send_sem=send_sems.at[d],
                recv_sem=recv_sems.at[my],
                device_id=(t,),
                device_id_type=pl.DeviceIdType.MESH,
            )
            rdma.start()
            sends.append(rdma)

        acc = jnp.zeros((MP, N), jnp.float32)
        for s in range(N_DEV):
            recv = pltpu.make_async_remote_copy(
                src_ref=xbf_ref.at[pl.ds(0, MP), :],
                dst_ref=xg_ref.at[s],
                send_sem=send_sems.at[0],
                recv_sem=recv_sems.at[s],
                device_id=(my,),
                device_id_type=pl.DeviceIdType.MESH,
            )
            recv.wait_recv()
            acc = acc + jnp.dot(
                xg_ref[s],
                w_ref[pl.ds(s * KP, KP), :].astype(jnp.bfloat16),
                preferred_element_type=jnp.float32,
            )

        for rdma in sends:
            rdma.wait_send()

        out_ref[:, :] = acc * jax.nn.sigmoid(acc)

    return pl.pallas_call(
        body,
        out_shape=jax.ShapeDtypeStruct((MP, N), jnp.float32),
        in_specs=[
            pl.BlockSpec(memory_space=pltpu.VMEM),
            pl.BlockSpec(memory_space=pltpu.VMEM),
        ],
        out_specs=pl.BlockSpec(memory_space=pltpu.VMEM),
        scratch_shapes=[
            pltpu.VMEM((M, kp), jnp.bfloat16),
            pltpu.VMEM((N_DEV, MP, KP), jnp.bfloat16),
            pltpu.SemaphoreType.DMA((N_DEV,)),
            pltpu.SemaphoreType.DMA((N_DEV,)),
        ],
    )(x, w_mat)
